# Initial kernel scaffold; baseline (speedup 1.0000x reference)
import jax
import jax.numpy as jnp
from jax.experimental import pallas as pl


def kernel(vectors, segment_ids, frag_table, site_table, bias):
    raise NotImplementedError("write your pallas kernel here")


# speedup vs baseline: 2.8019x; 2.8019x over previous
import functools

import jax
import jax.numpy as jnp
from jax import lax
from jax.experimental import pallas as pl
from jax.experimental.pallas import tpu as pltpu
from jax.experimental.pallas import tpu_sc as plsc

TOTAL = 32768
NSEG = 16
NC = 2
NS = 16
NW = NC * NS
ROWS = 8
COLS = 128
PER_TILE = ROWS * COLS


@functools.partial(
    pl.kernel,
    out_type=(
        jax.ShapeDtypeStruct((NW, NSEG), jnp.float32),
        jax.ShapeDtypeStruct((NW, NSEG), jnp.float32),
    ),
    mesh=plsc.VectorSubcoreMesh(
        core_axis_name="c", subcore_axis_name="s", num_cores=NC, num_subcores=NS
    ),
    compiler_params=pltpu.CompilerParams(needs_layout_passes=False),
    scratch_types=(
        pltpu.VMEM((ROWS, COLS), jnp.int32),
        pltpu.VMEM((ROWS, COLS), jnp.int32),
        pltpu.VMEM((ROWS, COLS), jnp.int32),
        pltpu.VMEM((ROWS, COLS), jnp.int32),
        pltpu.VMEM((ROWS, COLS), jnp.float32),
        pltpu.VMEM((ROWS, COLS), jnp.float32),
        pltpu.VMEM((NSEG,), jnp.float32),
        pltpu.VMEM((NSEG,), jnp.float32),
        pltpu.SemaphoreType.DMA,
    ),
)
def _sc_pool(fidx_hbm, sidx_hbm, seg_hbm, segn_hbm, ftab_hbm, stab_hbm,
             num_hbm, den_hbm,
             fidx_v, sidx_v, seg_v, segn_v, fval_v, sval_v,
             accn_v, accd_v, sem):
    wid = lax.axis_index("s") * NC + lax.axis_index("c")

    pltpu.sync_copy(fidx_hbm.at[wid], fidx_v)
    pltpu.sync_copy(sidx_hbm.at[wid], sidx_v)
    pltpu.sync_copy(seg_hbm.at[wid], seg_v)
    pltpu.sync_copy(segn_hbm.at[wid], segn_v)

    copies = []
    for j in range(ROWS):
        copies.append(pltpu.async_copy(ftab_hbm.at[fidx_v.at[j]], fval_v.at[j], sem))
        copies.append(pltpu.async_copy(stab_hbm.at[sidx_v.at[j]], sval_v.at[j], sem))
    for cp in copies:
        cp.wait()

    accn_v[...] = jnp.zeros((NSEG,), jnp.float32)
    accd_v[...] = jnp.zeros((NSEG,), jnp.float32)

    lane = lax.iota(jnp.int32, 16)
    lane_lt15 = lane < 15
    lane_eq15 = lane == 15

    for j in range(ROWS):
        for k in range(COLS // 16):
            sl = pl.ds(k * 16, 16)
            f = fval_v[j, sl]
            s = sval_v[j, sl]
            g = seg_v[j, sl]
            gn = segn_v[j, sl]
            a = jnp.exp(f)
            w = a * s
            ca = plsc.cumsum(a)
            cw = plsc.cumsum(w)
            m = g != gn
            mf = m | lane_eq15
            mm = m & lane_lt15
            plsc.addupdate_scatter(accd_v, [g], ca, mask=mf)
            plsc.addupdate_scatter(accn_v, [g], cw, mask=mf)
            plsc.addupdate_scatter(accd_v, [gn], -ca, mask=mm)
            plsc.addupdate_scatter(accn_v, [gn], -cw, mask=mm)

    pltpu.sync_copy(accn_v, num_hbm.at[wid])
    pltpu.sync_copy(accd_v, den_hbm.at[wid])


def _combine_body(num_ref, den_ref, bias_ref, out_ref):
    num = jnp.sum(num_ref[...], axis=0)
    den = jnp.sum(den_ref[...], axis=0) + 0.001
    out_ref[...] = num / den + bias_ref[0]


_combine = pl.pallas_call(
    _combine_body,
    out_shape=jax.ShapeDtypeStruct((NSEG,), jnp.float32),
)


def kernel(vectors, segment_ids, frag_table, site_table, bias):
    fidx = vectors[:, 1].reshape(NW, ROWS, COLS)
    sidx = vectors[:, 0].reshape(NW, ROWS, COLS)
    seg = segment_ids.reshape(NW, ROWS, COLS)
    segn = jnp.concatenate(
        [segment_ids[1:], jnp.full((1,), NSEG, jnp.int32)]
    ).reshape(NW, ROWS, COLS)
    ftab = frag_table[:, 0]
    stab = site_table[:, 0]
    num_parts, den_parts = _sc_pool(fidx, sidx, seg, segn, ftab, stab)
    return _combine(num_parts, den_parts, bias)
